# baseline (speedup 1.0000x reference)
import functools

import jax
import jax.numpy as jnp
from jax import lax
from jax.experimental import pallas as pl
from jax.experimental.pallas import tpu as pltpu
from jax.experimental.pallas import tpu_sc as plsc

B = 32
NPTS = 2048
N_VERTS = 2562
V_PAD = 2568
NG = 2208
SEGS = ((0, 600, 600, 0.45), (600, 704, 700, 0.45),
        (1304, 400, 400, 0.05), (1704, 504, 500, 0.05))
N_REAL = 2200


def _sc_gather_body(verts_hbm, idx_hbm, out_hbm, vref, iref, gref):
    c = lax.axis_index("c")
    s = lax.axis_index("s")
    b = s * 2 + c
    pltpu.sync_copy(verts_hbm.at[b], vref)
    pltpu.sync_copy(idx_hbm, iref)

    def body(j, carry):
        iv = iref[pl.ds(j * 16, 16)]
        for ch in range(3):
            vals = plsc.load_gather(vref, [iv + ch * V_PAD])
            gref[pl.ds(ch * NG + j * 16, 16)] = vals
        return carry

    lax.fori_loop(0, NG // 16, body, 0)
    pltpu.sync_copy(gref, out_hbm.at[b])


@functools.lru_cache(maxsize=1)
def _sc_gather():
    return pl.kernel(
        _sc_gather_body,
        out_type=jax.ShapeDtypeStruct((B, 3 * NG), jnp.float32),
        mesh=plsc.VectorSubcoreMesh(core_axis_name="c", subcore_axis_name="s"),
        compiler_params=pltpu.CompilerParams(needs_layout_passes=False),
        scratch_types=[
            pltpu.VMEM((3 * V_PAD,), jnp.float32),
            pltpu.VMEM((NG,), jnp.int32),
            pltpu.VMEM((3 * NG,), jnp.float32),
        ],
    )


def _tc_body(g_ref, cams_ref, hp_ref, bp_ref, np_ref, kp_ref, v2d_ref, loss_ref):
    sc = cams_ref[0, 0, 0]
    tx = cams_ref[0, 0, 1]
    ty = cams_ref[0, 0, 2]
    qw = cams_ref[0, 0, 3]
    qx = cams_ref[0, 0, 4]
    qy = cams_ref[0, 0, 5]
    qz = cams_ref[0, 0, 6]
    n2 = qw * qw + qx * qx + qy * qy + qz * qz
    r = lax.rsqrt(n2)
    r = r * (1.5 - 0.5 * n2 * r * r)
    r = r * (1.5 - 0.5 * n2 * r * r)
    den = n2 * r + 1e-8
    inv = r * (2.0 - den * r)
    inv = inv * (2.0 - den * inv)
    inv = inv * (2.0 - den * inv)
    w = qw * inv
    x = qx * inv
    y = qy * inv
    z = qz * inv
    def bf(v):
        return lax.convert_element_type(
            lax.convert_element_type(v, jnp.bfloat16), jnp.float32)

    b00 = bf(1.0 - 2.0 * (y * y + z * z))
    b01 = bf(2.0 * (x * y - w * z))
    b02 = bf(2.0 * (x * z + w * y))
    b10 = bf(2.0 * (x * y + w * z))
    b11 = bf(1.0 - 2.0 * (x * x + z * z))
    b12 = bf(2.0 * (y * z - w * x))

    def chunk(v0, pref):
        gx = bf(g_ref[0, pl.ds(v0, 8), 0:1])
        gy = bf(g_ref[0, pl.ds(v0, 8), 1:2])
        gz = bf(g_ref[0, pl.ds(v0, 8), 2:3])
        vx = sc * (b00 * gx + b01 * gy + b02 * gz) + tx
        vy = sc * (b10 * gx + b11 * gy + b12 * gz) + ty
        v2d_ref[0, pl.ds(v0, 8), 0:1] = vx
        v2d_ref[0, pl.ds(v0, 8), 1:2] = vy
        aa = vx * vx + vy * vy
        vxb2 = 2.0 * bf(vx)
        vyb2 = 2.0 * bf(vy)
        dm = None
        for pc in range(2):
            px = pref[0, 0:1, pc * 1024:(pc + 1) * 1024]
            py = pref[0, 1:2, pc * 1024:(pc + 1) * 1024]
            app = aa + (px * px + py * py)
            ab2 = vxb2 * bf(px) + vyb2 * bf(py)
            d = app - ab2
            dm = d if dm is None else jnp.minimum(dm, d)
        return jnp.min(dm, axis=1, keepdims=True)

    total = jnp.zeros((8, 1), jnp.float32)
    for (off, _lpad, lreal, wgt), pref in zip(SEGS, (hp_ref, bp_ref, np_ref, kp_ref)):
        nfull = lreal // 8

        def body(i, a, off=off, pref=pref, wgt=wgt):
            return a + wgt * chunk(off + i * 8, pref)

        total = lax.fori_loop(0, nfull, body, total)
        tail = lreal % 8
        if tail:
            dm = chunk(off + nfull * 8, pref)
            mask = lax.broadcasted_iota(jnp.int32, (8, 1), 0) < tail
            total = total + wgt * jnp.where(mask, dm, 0.0)
    loss_ref[0, :, :] = total


def _tc_call(g2, cams, hp, bp, npp, kp):
    return pl.pallas_call(
        _tc_body,
        grid=(B,),
        in_specs=[
            pl.BlockSpec((1, NG, 3), lambda b: (b, 0, 0)),
            pl.BlockSpec((1, 1, 7), lambda b: (b, 0, 0), memory_space=pltpu.SMEM),
            pl.BlockSpec((1, 2, NPTS), lambda b: (b, 0, 0)),
            pl.BlockSpec((1, 2, NPTS), lambda b: (b, 0, 0)),
            pl.BlockSpec((1, 2, NPTS), lambda b: (b, 0, 0)),
            pl.BlockSpec((1, 2, NPTS), lambda b: (b, 0, 0)),
        ],
        out_specs=[
            pl.BlockSpec((1, NG, 2), lambda b: (b, 0, 0)),
            pl.BlockSpec((1, 8, 1), lambda b: (b, 0, 0)),
        ],
        out_shape=[
            jax.ShapeDtypeStruct((B, NG, 2), jnp.float32),
            jax.ShapeDtypeStruct((B, 8, 1), jnp.float32),
        ],
    )(g2, cams, hp, bp, npp, kp)


def kernel(head_points, belly_points, neck_points, back_points, verts, cams,
           head_idx, belly_idx, neck_idx, back_idx):
    z4 = jnp.zeros((4,), jnp.int32)
    idx_all = jnp.concatenate([head_idx, belly_idx, z4, neck_idx, back_idx, z4])
    verts_t = jnp.pad(verts.transpose(0, 2, 1),
                      ((0, 0), (0, 0), (0, V_PAD - N_VERTS))).reshape(B, 3 * V_PAD)
    g = _sc_gather()(verts_t, idx_all)
    g2 = g.reshape(B, 3, NG).transpose(0, 2, 1)
    hp = head_points.transpose(0, 2, 1)
    bp = belly_points.transpose(0, 2, 1)
    npp = neck_points.transpose(0, 2, 1)
    kp = back_points.transpose(0, 2, 1)
    v2d_pad, lp = _tc_call(g2, cams.reshape(B, 1, 7), hp, bp, npp, kp)
    vert2d = jnp.concatenate([
        v2d_pad[:, 0:600], v2d_pad[:, 600:1300],
        v2d_pad[:, 1304:1704], v2d_pad[:, 1704:2204]], axis=1)
    loss = jnp.sum(lp) / (B * float(N_REAL))
    return loss, vert2d


# speedup vs baseline: 3.0859x; 3.0859x over previous
import functools

import jax
import jax.numpy as jnp
from jax import lax
from jax.experimental import pallas as pl
from jax.experimental.pallas import tpu as pltpu
from jax.experimental.pallas import tpu_sc as plsc

B = 32
NPTS = 2048
N_VERTS = 2562
V_PAD = 2568
NG = 2240
SEGS = ((0, 608, 600, 0.45, 19), (608, 704, 700, 0.45, 22),
        (1312, 416, 400, 0.05, 13), (1728, 512, 500, 0.05, 16))
N_REAL = 2200
PC = 512


def _sc_gather_body(verts_hbm, idx_hbm, out_hbm, vref, iref, gref):
    c = lax.axis_index("c")
    s = lax.axis_index("s")
    b = s * 2 + c
    pltpu.sync_copy(verts_hbm.at[b], vref)
    pltpu.sync_copy(idx_hbm, iref)

    def body(j, carry):
        iv = iref[pl.ds(j * 16, 16)]
        for ch in range(3):
            vals = plsc.load_gather(vref, [iv + ch * V_PAD])
            gref[pl.ds(ch * NG + j * 16, 16)] = vals
        return carry

    lax.fori_loop(0, NG // 16, body, 0)
    pltpu.sync_copy(gref, out_hbm.at[b])


@functools.lru_cache(maxsize=1)
def _sc_gather():
    return pl.kernel(
        _sc_gather_body,
        out_type=jax.ShapeDtypeStruct((B, 3 * NG), jnp.float32),
        mesh=plsc.VectorSubcoreMesh(core_axis_name="c", subcore_axis_name="s"),
        compiler_params=pltpu.CompilerParams(needs_layout_passes=False),
        scratch_types=[
            pltpu.VMEM((3 * V_PAD,), jnp.float32),
            pltpu.VMEM((NG,), jnp.int32),
            pltpu.VMEM((3 * NG,), jnp.float32),
        ],
    )


def _bf(v):
    return lax.convert_element_type(
        lax.convert_element_type(v, jnp.bfloat16), jnp.float32)


def _tc_body(g_ref, cams_ref, wv_ref, hp_ref, bp_ref, np_ref, kp_ref,
             v2d_ref, loss_ref, scr_ref):
    sc = cams_ref[0, 0, 0]
    tx = cams_ref[0, 0, 1]
    ty = cams_ref[0, 0, 2]
    qw = cams_ref[0, 0, 3]
    qx = cams_ref[0, 0, 4]
    qy = cams_ref[0, 0, 5]
    qz = cams_ref[0, 0, 6]
    n2 = qw * qw + qx * qx + qy * qy + qz * qz
    r = lax.rsqrt(n2)
    r = r * (1.5 - 0.5 * n2 * r * r)
    r = r * (1.5 - 0.5 * n2 * r * r)
    den = n2 * r + 1e-8
    inv = r * (2.0 - den * r)
    inv = inv * (2.0 - den * inv)
    inv = inv * (2.0 - den * inv)
    w = qw * inv
    x = qx * inv
    y = qy * inv
    z = qz * inv
    b00 = _bf(1.0 - 2.0 * (y * y + z * z))
    b01 = _bf(2.0 * (x * y - w * z))
    b02 = _bf(2.0 * (x * z + w * y))
    b10 = _bf(2.0 * (x * y + w * z))
    b11 = _bf(1.0 - 2.0 * (x * x + z * z))
    b12 = _bf(2.0 * (y * z - w * x))

    total = jnp.zeros((32, 1), jnp.float32)
    for (off, _lpad, _lreal, _wgt, nchunks), pref in zip(
            SEGS, (hp_ref, bp_ref, np_ref, kp_ref)):
        px = pref[0, 0:1, :]
        py = pref[0, 1:2, :]
        scr_ref[0:1, :] = px * px + py * py
        scr_ref[1:2, :] = _bf(px)
        scr_ref[2:3, :] = _bf(py)

        def body(i, acc, off=off, nchunks=nchunks):
            v0 = off + i * 32
            gx = _bf(g_ref[0, pl.ds(v0, 32), 0:1])
            gy = _bf(g_ref[0, pl.ds(v0, 32), 1:2])
            gz = _bf(g_ref[0, pl.ds(v0, 32), 2:3])
            vx = sc * (b00 * gx + b01 * gy + b02 * gz) + tx
            vy = sc * (b10 * gx + b11 * gy + b12 * gz) + ty
            v2d_ref[0, pl.ds(v0, 32), 0:1] = vx
            v2d_ref[0, pl.ds(v0, 32), 1:2] = vy
            aa = vx * vx + vy * vy
            vxb2 = 2.0 * _bf(vx)
            vyb2 = 2.0 * _bf(vy)
            dm = None
            for pc in range(NPTS // PC):
                pp = scr_ref[0:1, pc * PC:(pc + 1) * PC]
                pxb = scr_ref[1:2, pc * PC:(pc + 1) * PC]
                pyb = scr_ref[2:3, pc * PC:(pc + 1) * PC]
                t = pp - vxb2 * pxb - vyb2 * pyb
                dm = t if dm is None else jnp.minimum(dm, t)
            dmin = jnp.min(dm, axis=1, keepdims=True) + aa
            wc = wv_ref[pl.ds(v0, 32), 0:1]
            return acc + wc * dmin

        total = lax.fori_loop(0, nchunks, body, total)
    loss_ref[0, :, :] = total


def _tc_call(g2, cams, wv, hp, bp, npp, kp):
    return pl.pallas_call(
        _tc_body,
        grid=(B,),
        in_specs=[
            pl.BlockSpec((1, NG, 3), lambda b: (b, 0, 0)),
            pl.BlockSpec((1, 1, 7), lambda b: (b, 0, 0),
                         memory_space=pltpu.SMEM),
            pl.BlockSpec((NG, 1), lambda b: (0, 0)),
            pl.BlockSpec((1, 2, NPTS), lambda b: (b, 0, 0)),
            pl.BlockSpec((1, 2, NPTS), lambda b: (b, 0, 0)),
            pl.BlockSpec((1, 2, NPTS), lambda b: (b, 0, 0)),
            pl.BlockSpec((1, 2, NPTS), lambda b: (b, 0, 0)),
        ],
        out_specs=[
            pl.BlockSpec((1, NG, 2), lambda b: (b, 0, 0)),
            pl.BlockSpec((1, 32, 1), lambda b: (b, 0, 0)),
        ],
        out_shape=[
            jax.ShapeDtypeStruct((B, NG, 2), jnp.float32),
            jax.ShapeDtypeStruct((B, 32, 1), jnp.float32),
        ],
        scratch_shapes=[pltpu.VMEM((3, NPTS), jnp.float32)],
    )(g2, cams, wv, hp, bp, npp, kp)


def kernel(head_points, belly_points, neck_points, back_points, verts, cams,
           head_idx, belly_idx, neck_idx, back_idx):
    def z(k):
        return jnp.zeros((k,), jnp.int32)

    idx_all = jnp.concatenate([head_idx, z(8), belly_idx, z(4),
                               neck_idx, z(16), back_idx, z(12)])
    wv = jnp.concatenate([
        jnp.full((600,), 0.45, jnp.float32), jnp.zeros((8,), jnp.float32),
        jnp.full((700,), 0.45, jnp.float32), jnp.zeros((4,), jnp.float32),
        jnp.full((400,), 0.05, jnp.float32), jnp.zeros((16,), jnp.float32),
        jnp.full((500,), 0.05, jnp.float32), jnp.zeros((12,), jnp.float32),
    ]).reshape(NG, 1)
    verts_t = jnp.pad(verts.transpose(0, 2, 1),
                      ((0, 0), (0, 0), (0, V_PAD - N_VERTS))).reshape(B, 3 * V_PAD)
    g = _sc_gather()(verts_t, idx_all)
    g2 = g.reshape(B, 3, NG).transpose(0, 2, 1)
    hp = head_points.transpose(0, 2, 1)
    bp = belly_points.transpose(0, 2, 1)
    npp = neck_points.transpose(0, 2, 1)
    kp = back_points.transpose(0, 2, 1)
    v2d_pad, lp = _tc_call(g2, cams.reshape(B, 1, 7), wv, hp, bp, npp, kp)
    vert2d = jnp.concatenate([
        v2d_pad[:, 0:600], v2d_pad[:, 608:1308],
        v2d_pad[:, 1312:1712], v2d_pad[:, 1728:2228]], axis=1)
    loss = jnp.sum(lp) / (B * float(N_REAL))
    return loss, vert2d


# speedup vs baseline: 5.9685x; 1.9341x over previous
import functools

import jax
import jax.numpy as jnp
from jax import lax
from jax.experimental import pallas as pl
from jax.experimental.pallas import tpu as pltpu
from jax.experimental.pallas import tpu_sc as plsc

B = 32
NPTS = 2048
N_VERTS = 2562
V_PAD = 2568
NG = 2432
SEGS = ((0, 640, 600, 0.45, 5), (640, 768, 700, 0.45, 6),
        (1408, 512, 400, 0.05, 4), (1920, 512, 500, 0.05, 4))
N_REAL = 2200
PC = 128


def _sc_gather_body(verts_hbm, idx_hbm, out_hbm, vref, iref, gref):
    c = lax.axis_index("c")
    s = lax.axis_index("s")
    b = s * 2 + c
    pltpu.sync_copy(verts_hbm.at[b], vref)
    pltpu.sync_copy(idx_hbm, iref)

    def body(j, carry):
        iv = iref[pl.ds(j * 16, 16)]
        for ch in range(3):
            vals = plsc.load_gather(vref, [iv + ch * V_PAD])
            gref[pl.ds(ch * NG + j * 16, 16)] = vals
        return carry

    lax.fori_loop(0, NG // 16, body, 0)
    pltpu.sync_copy(gref, out_hbm.at[b])


@functools.lru_cache(maxsize=1)
def _sc_gather():
    return pl.kernel(
        _sc_gather_body,
        out_type=jax.ShapeDtypeStruct((B, 3 * NG), jnp.float32),
        mesh=plsc.VectorSubcoreMesh(core_axis_name="c", subcore_axis_name="s"),
        compiler_params=pltpu.CompilerParams(needs_layout_passes=False),
        scratch_types=[
            pltpu.VMEM((3 * V_PAD,), jnp.float32),
            pltpu.VMEM((NG,), jnp.int32),
            pltpu.VMEM((3 * NG,), jnp.float32),
        ],
    )


def _bf(v):
    return lax.convert_element_type(
        lax.convert_element_type(v, jnp.bfloat16), jnp.float32)


def _tc_body(g_ref, cams_ref, wv_ref, hp_ref, bp_ref, np_ref, kp_ref,
             v2d_ref, loss_ref, scr_ref):
    sc = cams_ref[0, 0, 0]
    tx = cams_ref[0, 0, 1]
    ty = cams_ref[0, 0, 2]
    qw = cams_ref[0, 0, 3]
    qx = cams_ref[0, 0, 4]
    qy = cams_ref[0, 0, 5]
    qz = cams_ref[0, 0, 6]
    n2 = qw * qw + qx * qx + qy * qy + qz * qz
    r = lax.rsqrt(n2)
    r = r * (1.5 - 0.5 * n2 * r * r)
    r = r * (1.5 - 0.5 * n2 * r * r)
    den = n2 * r + 1e-8
    inv = r * (2.0 - den * r)
    inv = inv * (2.0 - den * inv)
    inv = inv * (2.0 - den * inv)
    w = qw * inv
    x = qx * inv
    y = qy * inv
    z = qz * inv
    b00 = _bf(1.0 - 2.0 * (y * y + z * z))
    b01 = _bf(2.0 * (x * y - w * z))
    b02 = _bf(2.0 * (x * z + w * y))
    b10 = _bf(2.0 * (x * y + w * z))
    b11 = _bf(1.0 - 2.0 * (x * x + z * z))
    b12 = _bf(2.0 * (y * z - w * x))

    total = jnp.zeros((32, 1), jnp.float32)
    for (off, _lpad, _lreal, _wgt, ngroups), pref in zip(
            SEGS, (hp_ref, bp_ref, np_ref, kp_ref)):
        px = pref[0, 0:1, :]
        py = pref[0, 1:2, :]
        pp8 = jnp.broadcast_to(px * px + py * py, (8, NPTS))
        pxb8 = jnp.broadcast_to(_bf(px), (8, NPTS))
        pyb8 = jnp.broadcast_to(_bf(py), (8, NPTS))
        for rr in range(4):
            scr_ref[0, pl.ds(rr * 8, 8), :] = pp8
            scr_ref[1, pl.ds(rr * 8, 8), :] = pxb8
            scr_ref[2, pl.ds(rr * 8, 8), :] = pyb8

        def group(i, acc, off=off):
            g0 = off + i * 128
            aa = []
            vxb = []
            vyb = []
            for u in range(4):
                v0 = g0 + u * 32
                gx = _bf(g_ref[0, pl.ds(v0, 32), 0:1])
                gy = _bf(g_ref[0, pl.ds(v0, 32), 1:2])
                gz = _bf(g_ref[0, pl.ds(v0, 32), 2:3])
                vx = sc * (b00 * gx + b01 * gy + b02 * gz) + tx
                vy = sc * (b10 * gx + b11 * gy + b12 * gz) + ty
                v2d_ref[0, pl.ds(v0, 32), 0:1] = vx
                v2d_ref[0, pl.ds(v0, 32), 1:2] = vy
                aa.append(vx * vx + vy * vy)
                vxb.append(jnp.broadcast_to(2.0 * _bf(vx), (32, PC)))
                vyb.append(jnp.broadcast_to(2.0 * _bf(vy), (32, PC)))
            dm = [None] * 4
            for pc in range(NPTS // PC):
                pp = scr_ref[0, :, pc * PC:(pc + 1) * PC]
                pxb = scr_ref[1, :, pc * PC:(pc + 1) * PC]
                pyb = scr_ref[2, :, pc * PC:(pc + 1) * PC]
                for u in range(4):
                    t = pp - vxb[u] * pxb - vyb[u] * pyb
                    dm[u] = t if dm[u] is None else jnp.minimum(dm[u], t)
            for u in range(4):
                dmin = jnp.min(dm[u], axis=1, keepdims=True) + aa[u]
                wc = wv_ref[pl.ds(g0 + u * 32, 32), 0:1]
                acc = acc + wc * dmin
            return acc

        total = lax.fori_loop(0, ngroups, group, total)
    loss_ref[0, :, :] = total


def _tc_call(g2, cams, wv, hp, bp, npp, kp):
    return pl.pallas_call(
        _tc_body,
        grid=(B,),
        in_specs=[
            pl.BlockSpec((1, NG, 3), lambda b: (b, 0, 0)),
            pl.BlockSpec((1, 1, 7), lambda b: (b, 0, 0),
                         memory_space=pltpu.SMEM),
            pl.BlockSpec((NG, 1), lambda b: (0, 0)),
            pl.BlockSpec((1, 2, NPTS), lambda b: (b, 0, 0)),
            pl.BlockSpec((1, 2, NPTS), lambda b: (b, 0, 0)),
            pl.BlockSpec((1, 2, NPTS), lambda b: (b, 0, 0)),
            pl.BlockSpec((1, 2, NPTS), lambda b: (b, 0, 0)),
        ],
        out_specs=[
            pl.BlockSpec((1, NG, 2), lambda b: (b, 0, 0)),
            pl.BlockSpec((1, 32, 1), lambda b: (b, 0, 0)),
        ],
        out_shape=[
            jax.ShapeDtypeStruct((B, NG, 2), jnp.float32),
            jax.ShapeDtypeStruct((B, 32, 1), jnp.float32),
        ],
        scratch_shapes=[pltpu.VMEM((3, 32, NPTS), jnp.float32)],
    )(g2, cams, wv, hp, bp, npp, kp)


def kernel(head_points, belly_points, neck_points, back_points, verts, cams,
           head_idx, belly_idx, neck_idx, back_idx):
    def z(k):
        return jnp.zeros((k,), jnp.int32)

    idx_all = jnp.concatenate([head_idx, z(40), belly_idx, z(68),
                               neck_idx, z(112), back_idx, z(12)])
    wv = jnp.concatenate([
        jnp.full((600,), 0.45, jnp.float32), jnp.zeros((40,), jnp.float32),
        jnp.full((700,), 0.45, jnp.float32), jnp.zeros((68,), jnp.float32),
        jnp.full((400,), 0.05, jnp.float32), jnp.zeros((112,), jnp.float32),
        jnp.full((500,), 0.05, jnp.float32), jnp.zeros((12,), jnp.float32),
    ]).reshape(NG, 1)
    verts_t = jnp.pad(verts.transpose(0, 2, 1),
                      ((0, 0), (0, 0), (0, V_PAD - N_VERTS))).reshape(B, 3 * V_PAD)
    g = _sc_gather()(verts_t, idx_all)
    g2 = g.reshape(B, 3, NG).transpose(0, 2, 1)
    hp = head_points.transpose(0, 2, 1)
    bp = belly_points.transpose(0, 2, 1)
    npp = neck_points.transpose(0, 2, 1)
    kp = back_points.transpose(0, 2, 1)
    v2d_pad, lp = _tc_call(g2, cams.reshape(B, 1, 7), wv, hp, bp, npp, kp)
    vert2d = jnp.concatenate([
        v2d_pad[:, 0:600], v2d_pad[:, 640:1340],
        v2d_pad[:, 1408:1808], v2d_pad[:, 1920:2420]], axis=1)
    loss = jnp.sum(lp) / (B * float(N_REAL))
    return loss, vert2d


# speedup vs baseline: 15.9312x; 2.6692x over previous
import functools

import jax
import jax.numpy as jnp
from jax import lax
from jax.experimental import pallas as pl
from jax.experimental.pallas import tpu as pltpu
from jax.experimental.pallas import tpu_sc as plsc

B = 32
NPTS = 2048
N_VERTS = 2562
V_PAD = 2568
NG = 2560
SEGS = ((0, 768, 600, 0.45), (768, 768, 700, 0.45),
        (1536, 512, 400, 0.05), (2048, 512, 500, 0.05))
N_REAL = 2200
KP = 24
VT = 256
MT = 256


def _sc_gather_body(verts_hbm, idx_hbm, out_hbm, vref, iref, gref):
    c = lax.axis_index("c")
    s = lax.axis_index("s")
    b = s * 2 + c
    pltpu.sync_copy(verts_hbm.at[b], vref)
    pltpu.sync_copy(idx_hbm, iref)

    def body(j, carry):
        iv = iref[pl.ds(j * 16, 16)]
        for ch in range(3):
            vals = plsc.load_gather(vref, [iv + ch * V_PAD])
            gref[pl.ds(ch * NG + j * 16, 16)] = vals
        return carry

    lax.fori_loop(0, NG // 16, body, 0)
    pltpu.sync_copy(gref, out_hbm.at[b])


@functools.lru_cache(maxsize=1)
def _sc_gather():
    return pl.kernel(
        _sc_gather_body,
        out_type=jax.ShapeDtypeStruct((B, 3 * NG), jnp.float32),
        mesh=plsc.VectorSubcoreMesh(core_axis_name="c", subcore_axis_name="s"),
        compiler_params=pltpu.CompilerParams(needs_layout_passes=False),
        scratch_types=[
            pltpu.VMEM((3 * V_PAD,), jnp.float32),
            pltpu.VMEM((NG,), jnp.int32),
            pltpu.VMEM((3 * NG,), jnp.float32),
        ],
    )


def _bf(v):
    return lax.convert_element_type(
        lax.convert_element_type(v, jnp.bfloat16), jnp.float32)


def _tc_body(g_ref, cams_ref, wv_ref, hp_ref, bp_ref, np_ref, kp_ref,
             v2d_ref, loss_ref, pv_ref, pm_ref, aa_ref):
    sc = cams_ref[0, 0, 0]
    tx = cams_ref[0, 0, 1]
    ty = cams_ref[0, 0, 2]
    qw = cams_ref[0, 0, 3]
    qx = cams_ref[0, 0, 4]
    qy = cams_ref[0, 0, 5]
    qz = cams_ref[0, 0, 6]
    n2 = qw * qw + qx * qx + qy * qy + qz * qz
    r = lax.rsqrt(n2)
    r = r * (1.5 - 0.5 * n2 * r * r)
    r = r * (1.5 - 0.5 * n2 * r * r)
    den = n2 * r + 1e-8
    inv = r * (2.0 - den * r)
    inv = inv * (2.0 - den * inv)
    inv = inv * (2.0 - den * inv)
    w = qw * inv
    x = qx * inv
    y = qy * inv
    z = qz * inv
    b00 = _bf(1.0 - 2.0 * (y * y + z * z))
    b01 = _bf(2.0 * (x * y - w * z))
    b02 = _bf(2.0 * (x * z + w * y))
    b10 = _bf(2.0 * (x * y + w * z))
    b11 = _bf(1.0 - 2.0 * (x * x + z * z))
    b12 = _bf(2.0 * (y * z - w * x))

    gx = _bf(g_ref[0, 0:1, :])
    gy = _bf(g_ref[0, 1:2, :])
    gz = _bf(g_ref[0, 2:3, :])
    vx = sc * (b00 * gx + b01 * gy + b02 * gz) + tx
    vy = sc * (b10 * gx + b11 * gy + b12 * gz) + ty
    v2d_ref[0, 0:1, :] = vx
    v2d_ref[0, 1:2, :] = vy
    aa_ref[0:1, :] = vx * vx + vy * vy

    lane = lax.broadcasted_iota(jnp.int32, (1, NG), 1)
    bx2 = -2.0 * _bf(vx)
    by2 = -2.0 * _bf(vy)
    one = jnp.ones((1, NG), jnp.float32)
    zero = jnp.zeros((1, NG), jnp.bfloat16)
    for si, (off, lpad, _lreal, _wgt) in enumerate(SEGS):
        m = (lane >= off) & (lane < off + lpad)
        pv_ref[5 * si + 0:5 * si + 1, :] = jnp.where(m, bx2, 0.0).astype(jnp.bfloat16)
        pv_ref[5 * si + 1:5 * si + 2, :] = jnp.where(m, by2, 0.0).astype(jnp.bfloat16)
        mb = jnp.where(m, one, 0.0).astype(jnp.bfloat16)
        pv_ref[5 * si + 2:5 * si + 3, :] = mb
        pv_ref[5 * si + 3:5 * si + 4, :] = mb
        pv_ref[5 * si + 4:5 * si + 5, :] = mb
    for k in range(20, KP):
        pv_ref[k:k + 1, :] = zero

    zp = jnp.zeros((1, NPTS), jnp.bfloat16)
    for si, pref in enumerate((hp_ref, bp_ref, np_ref, kp_ref)):
        px = pref[0, 0:1, :]
        py = pref[0, 1:2, :]
        pp = px * px + py * py
        hh = _bf(pp)
        hm = _bf(pp - hh)
        hl = pp - hh - hm
        pm_ref[5 * si + 0:5 * si + 1, :] = px.astype(jnp.bfloat16)
        pm_ref[5 * si + 1:5 * si + 2, :] = py.astype(jnp.bfloat16)
        pm_ref[5 * si + 2:5 * si + 3, :] = hh.astype(jnp.bfloat16)
        pm_ref[5 * si + 3:5 * si + 4, :] = hm.astype(jnp.bfloat16)
        pm_ref[5 * si + 4:5 * si + 5, :] = hl.astype(jnp.bfloat16)
    for k in range(20, KP):
        pm_ref[k:k + 1, :] = zp

    total = jnp.zeros((1, VT), jnp.float32)
    for vt in range(NG // VT):
        rv = pv_ref[:, vt * VT:(vt + 1) * VT]
        dm = None
        for mt in range(NPTS // MT):
            lp = pm_ref[:, mt * MT:(mt + 1) * MT]
            dt = lax.dot_general(lp, rv, (((0,), (0,)), ((), ())),
                                 preferred_element_type=jnp.float32)
            r1 = jnp.min(dt, axis=0, keepdims=True)
            dm = r1 if dm is None else jnp.minimum(dm, r1)
        aa = aa_ref[0:1, vt * VT:(vt + 1) * VT]
        wc = wv_ref[0:1, vt * VT:(vt + 1) * VT]
        total = total + wc * (dm + aa)
    loss_ref[0, 0:1, :] = total


def _tc_call(g2, cams, wv, hp, bp, npp, kp):
    return pl.pallas_call(
        _tc_body,
        grid=(B,),
        in_specs=[
            pl.BlockSpec((1, 3, NG), lambda b: (b, 0, 0)),
            pl.BlockSpec((1, 1, 7), lambda b: (b, 0, 0),
                         memory_space=pltpu.SMEM),
            pl.BlockSpec((1, NG), lambda b: (0, 0)),
            pl.BlockSpec((1, 2, NPTS), lambda b: (b, 0, 0)),
            pl.BlockSpec((1, 2, NPTS), lambda b: (b, 0, 0)),
            pl.BlockSpec((1, 2, NPTS), lambda b: (b, 0, 0)),
            pl.BlockSpec((1, 2, NPTS), lambda b: (b, 0, 0)),
        ],
        out_specs=[
            pl.BlockSpec((1, 2, NG), lambda b: (b, 0, 0)),
            pl.BlockSpec((1, 1, VT), lambda b: (b, 0, 0)),
        ],
        out_shape=[
            jax.ShapeDtypeStruct((B, 2, NG), jnp.float32),
            jax.ShapeDtypeStruct((B, 1, VT), jnp.float32),
        ],
        scratch_shapes=[
            pltpu.VMEM((KP, NG), jnp.bfloat16),
            pltpu.VMEM((KP, NPTS), jnp.bfloat16),
            pltpu.VMEM((1, NG), jnp.float32),
        ],
    )(g2, cams, wv, hp, bp, npp, kp)


def kernel(head_points, belly_points, neck_points, back_points, verts, cams,
           head_idx, belly_idx, neck_idx, back_idx):
    def z(k):
        return jnp.zeros((k,), jnp.int32)

    idx_all = jnp.concatenate([head_idx, z(168), belly_idx, z(68),
                               neck_idx, z(112), back_idx, z(12)])
    wv = jnp.concatenate([
        jnp.full((600,), 0.45, jnp.float32), jnp.zeros((168,), jnp.float32),
        jnp.full((700,), 0.45, jnp.float32), jnp.zeros((68,), jnp.float32),
        jnp.full((400,), 0.05, jnp.float32), jnp.zeros((112,), jnp.float32),
        jnp.full((500,), 0.05, jnp.float32), jnp.zeros((12,), jnp.float32),
    ]).reshape(1, NG)
    verts_t = jnp.pad(verts.transpose(0, 2, 1),
                      ((0, 0), (0, 0), (0, V_PAD - N_VERTS))).reshape(B, 3 * V_PAD)
    g = _sc_gather()(verts_t, idx_all)
    g2 = g.reshape(B, 3, NG)
    hp = head_points.transpose(0, 2, 1)
    bp = belly_points.transpose(0, 2, 1)
    npp = neck_points.transpose(0, 2, 1)
    kp = back_points.transpose(0, 2, 1)
    v2d_pad, lp = _tc_call(g2, cams.reshape(B, 1, 7), wv, hp, bp, npp, kp)
    vp = v2d_pad.transpose(0, 2, 1)
    vert2d = jnp.concatenate([
        vp[:, 0:600], vp[:, 768:1468],
        vp[:, 1536:1936], vp[:, 2048:2548]], axis=1)
    loss = jnp.sum(lp) / (B * float(N_REAL))
    return loss, vert2d


# speedup vs baseline: 17.0095x; 1.0677x over previous
import functools

import jax
import jax.numpy as jnp
from jax import lax
from jax.experimental import pallas as pl
from jax.experimental.pallas import tpu as pltpu
from jax.experimental.pallas import tpu_sc as plsc

B = 32
NPTS = 2048
N_VERTS = 2562
V_PAD = 2568
NG = 2304
SEGS = ((0, 608, 600, 0.45), (608, 704, 700, 0.45),
        (1312, 416, 400, 0.05), (1728, 576, 500, 0.05))
N_REAL = 2200
KP = 24
VT = 256
MT = 256


def _sc_gather_body(verts_hbm, idx_hbm, out_hbm, vref, iref, gref):
    c = lax.axis_index("c")
    s = lax.axis_index("s")
    b = s * 2 + c
    pltpu.sync_copy(verts_hbm.at[b], vref)
    pltpu.sync_copy(idx_hbm, iref)

    def body(j, carry):
        iv = iref[pl.ds(j * 16, 16)]
        for ch in range(3):
            vals = plsc.load_gather(vref, [iv + ch * V_PAD])
            gref[pl.ds(ch * NG + j * 16, 16)] = vals
        return carry

    lax.fori_loop(0, NG // 16, body, 0)
    pltpu.sync_copy(gref, out_hbm.at[b])


@functools.lru_cache(maxsize=1)
def _sc_gather():
    return pl.kernel(
        _sc_gather_body,
        out_type=jax.ShapeDtypeStruct((B, 3 * NG), jnp.float32),
        mesh=plsc.VectorSubcoreMesh(core_axis_name="c", subcore_axis_name="s"),
        compiler_params=pltpu.CompilerParams(needs_layout_passes=False),
        scratch_types=[
            pltpu.VMEM((3 * V_PAD,), jnp.float32),
            pltpu.VMEM((NG,), jnp.int32),
            pltpu.VMEM((3 * NG,), jnp.float32),
        ],
    )


def _bf(v):
    return lax.convert_element_type(
        lax.convert_element_type(v, jnp.bfloat16), jnp.float32)


def _tc_body(g_ref, cams_ref, wv_ref, hp_ref, bp_ref, np_ref, kp_ref,
             v2d_ref, loss_ref, pv_ref, pm_ref, aa_ref):
    sc = cams_ref[0, 0, 0]
    tx = cams_ref[0, 0, 1]
    ty = cams_ref[0, 0, 2]
    qw = cams_ref[0, 0, 3]
    qx = cams_ref[0, 0, 4]
    qy = cams_ref[0, 0, 5]
    qz = cams_ref[0, 0, 6]
    n2 = qw * qw + qx * qx + qy * qy + qz * qz
    r = lax.rsqrt(n2)
    r = r * (1.5 - 0.5 * n2 * r * r)
    r = r * (1.5 - 0.5 * n2 * r * r)
    den = n2 * r + 1e-8
    inv = r * (2.0 - den * r)
    inv = inv * (2.0 - den * inv)
    inv = inv * (2.0 - den * inv)
    w = qw * inv
    x = qx * inv
    y = qy * inv
    z = qz * inv
    b00 = _bf(1.0 - 2.0 * (y * y + z * z))
    b01 = _bf(2.0 * (x * y - w * z))
    b02 = _bf(2.0 * (x * z + w * y))
    b10 = _bf(2.0 * (x * y + w * z))
    b11 = _bf(1.0 - 2.0 * (x * x + z * z))
    b12 = _bf(2.0 * (y * z - w * x))

    gx = _bf(g_ref[0, 0:1, :])
    gy = _bf(g_ref[0, 1:2, :])
    gz = _bf(g_ref[0, 2:3, :])
    vx = sc * (b00 * gx + b01 * gy + b02 * gz) + tx
    vy = sc * (b10 * gx + b11 * gy + b12 * gz) + ty
    v2d_ref[0, 0:1, :] = vx
    v2d_ref[0, 1:2, :] = vy
    aa_ref[0:1, :] = vx * vx + vy * vy

    lane = lax.broadcasted_iota(jnp.int32, (1, NG), 1)
    bx2 = -2.0 * _bf(vx)
    by2 = -2.0 * _bf(vy)
    one = jnp.ones((1, NG), jnp.float32)
    zero = jnp.zeros((1, NG), jnp.bfloat16)
    for si, (off, lpad, _lreal, _wgt) in enumerate(SEGS):
        m = (lane >= off) & (lane < off + lpad)
        pv_ref[5 * si + 0:5 * si + 1, :] = jnp.where(m, bx2, 0.0).astype(jnp.bfloat16)
        pv_ref[5 * si + 1:5 * si + 2, :] = jnp.where(m, by2, 0.0).astype(jnp.bfloat16)
        mb = jnp.where(m, one, 0.0).astype(jnp.bfloat16)
        pv_ref[5 * si + 2:5 * si + 3, :] = mb
        pv_ref[5 * si + 3:5 * si + 4, :] = mb
        pv_ref[5 * si + 4:5 * si + 5, :] = mb
    for k in range(20, KP):
        pv_ref[k:k + 1, :] = zero

    zp = jnp.zeros((1, NPTS), jnp.bfloat16)
    for si, pref in enumerate((hp_ref, bp_ref, np_ref, kp_ref)):
        px = pref[0, 0:1, :]
        py = pref[0, 1:2, :]
        pp = px * px + py * py
        hh = _bf(pp)
        hm = _bf(pp - hh)
        hl = pp - hh - hm
        pm_ref[5 * si + 0:5 * si + 1, :] = px.astype(jnp.bfloat16)
        pm_ref[5 * si + 1:5 * si + 2, :] = py.astype(jnp.bfloat16)
        pm_ref[5 * si + 2:5 * si + 3, :] = hh.astype(jnp.bfloat16)
        pm_ref[5 * si + 3:5 * si + 4, :] = hm.astype(jnp.bfloat16)
        pm_ref[5 * si + 4:5 * si + 5, :] = hl.astype(jnp.bfloat16)
    for k in range(20, KP):
        pm_ref[k:k + 1, :] = zp

    total = jnp.zeros((1, VT), jnp.float32)
    for vt in range(NG // VT):
        rv = pv_ref[:, vt * VT:(vt + 1) * VT]
        dm = None
        for mt in range(NPTS // MT):
            lp = pm_ref[:, mt * MT:(mt + 1) * MT]
            dt = lax.dot_general(lp, rv, (((0,), (0,)), ((), ())),
                                 preferred_element_type=jnp.float32)
            r1 = jnp.min(dt, axis=0, keepdims=True)
            dm = r1 if dm is None else jnp.minimum(dm, r1)
        aa = aa_ref[0:1, vt * VT:(vt + 1) * VT]
        wc = wv_ref[0:1, vt * VT:(vt + 1) * VT]
        total = total + wc * (dm + aa)
    loss_ref[0, 0:1, :] = total


def _tc_call(g2, cams, wv, hp, bp, npp, kp):
    return pl.pallas_call(
        _tc_body,
        grid=(B,),
        in_specs=[
            pl.BlockSpec((1, 3, NG), lambda b: (b, 0, 0)),
            pl.BlockSpec((1, 1, 7), lambda b: (b, 0, 0),
                         memory_space=pltpu.SMEM),
            pl.BlockSpec((1, NG), lambda b: (0, 0)),
            pl.BlockSpec((1, 2, NPTS), lambda b: (b, 0, 0)),
            pl.BlockSpec((1, 2, NPTS), lambda b: (b, 0, 0)),
            pl.BlockSpec((1, 2, NPTS), lambda b: (b, 0, 0)),
            pl.BlockSpec((1, 2, NPTS), lambda b: (b, 0, 0)),
        ],
        out_specs=[
            pl.BlockSpec((1, 2, NG), lambda b: (b, 0, 0)),
            pl.BlockSpec((1, 1, VT), lambda b: (b, 0, 0)),
        ],
        out_shape=[
            jax.ShapeDtypeStruct((B, 2, NG), jnp.float32),
            jax.ShapeDtypeStruct((B, 1, VT), jnp.float32),
        ],
        scratch_shapes=[
            pltpu.VMEM((KP, NG), jnp.bfloat16),
            pltpu.VMEM((KP, NPTS), jnp.bfloat16),
            pltpu.VMEM((1, NG), jnp.float32),
        ],
    )(g2, cams, wv, hp, bp, npp, kp)


def kernel(head_points, belly_points, neck_points, back_points, verts, cams,
           head_idx, belly_idx, neck_idx, back_idx):
    def z(k):
        return jnp.zeros((k,), jnp.int32)

    idx_all = jnp.concatenate([head_idx, z(8), belly_idx, z(4),
                               neck_idx, z(16), back_idx, z(76)])
    wv = jnp.concatenate([
        jnp.full((600,), 0.45, jnp.float32), jnp.zeros((8,), jnp.float32),
        jnp.full((700,), 0.45, jnp.float32), jnp.zeros((4,), jnp.float32),
        jnp.full((400,), 0.05, jnp.float32), jnp.zeros((16,), jnp.float32),
        jnp.full((500,), 0.05, jnp.float32), jnp.zeros((76,), jnp.float32),
    ]).reshape(1, NG)
    verts_t = jnp.pad(verts.transpose(0, 2, 1),
                      ((0, 0), (0, 0), (0, V_PAD - N_VERTS))).reshape(B, 3 * V_PAD)
    g = _sc_gather()(verts_t, idx_all)
    g2 = g.reshape(B, 3, NG)
    hp = head_points.transpose(0, 2, 1)
    bp = belly_points.transpose(0, 2, 1)
    npp = neck_points.transpose(0, 2, 1)
    kp = back_points.transpose(0, 2, 1)
    v2d_pad, lp = _tc_call(g2, cams.reshape(B, 1, 7), wv, hp, bp, npp, kp)
    vp = v2d_pad.transpose(0, 2, 1)
    vert2d = jnp.concatenate([
        vp[:, 0:600], vp[:, 608:1308],
        vp[:, 1312:1712], vp[:, 1728:2228]], axis=1)
    loss = jnp.sum(lp) / (B * float(N_REAL))
    return loss, vert2d
